# baseline (speedup 1.0000x reference)
import functools

import jax
import jax.numpy as jnp
from jax import lax
from jax.experimental import pallas as pl
from jax.experimental.pallas import tpu as pltpu
from jax.experimental.pallas import tpu_sc as plsc

_HID = 64
_SCALE = 8.0

_NC = 2
_NS = 16
_NW = _NC * _NS
_LANES = 16

_CHUNK = 128
_VPC = _CHUNK * _HID // _LANES


def _make_kernel(batch, nchunk):
    b_per_w = nchunk * _CHUNK
    mesh = plsc.VectorSubcoreMesh(
        core_axis_name="c", subcore_axis_name="s",
        num_cores=_NC, num_subcores=_NS,
    )

    @functools.partial(
        pl.kernel,
        out_type=jax.ShapeDtypeStruct((batch, _HID), jnp.float32),
        mesh=mesh,
        scratch_types=[
            pltpu.VMEM((nchunk, _CHUNK), jnp.int32),
            pltpu.VMEM((_CHUNK, _HID), jnp.float32),
            pltpu.SemaphoreType.DMA,
        ],
        compiler_params=pltpu.CompilerParams(use_tc_tiling_on_sc=False),
    )
    def emb_kernel(table_hbm, idx_hbm, out_hbm, idx_v, rows_v, gsem):
        wid = lax.axis_index("s") * _NC + lax.axis_index("c")
        base = wid * b_per_w
        pltpu.sync_copy(idx_hbm.at[wid], idx_v)

        @pl.loop(0, nchunk)
        def _chunk_loop(g):
            pltpu.async_copy(table_hbm.at[idx_v.at[g]], rows_v, gsem).wait()

            @pl.loop(0, _CHUNK)
            def _scale_loop(r):
                for c in range(_HID // _LANES):
                    sl = pl.ds(c * _LANES, _LANES)
                    rows_v[r, sl] = rows_v[r, sl] * _SCALE

            pltpu.sync_copy(rows_v, out_hbm.at[pl.ds(base + g * _CHUNK, _CHUNK)])

    return emb_kernel


def kernel(x, emb_table):
    orig_shape = x.shape
    idx = x.reshape(-1).astype(jnp.int32)
    batch = idx.shape[0]
    assert batch % (_NW * _CHUNK) == 0
    nchunk = batch // (_NW * _CHUNK)
    idx3 = idx.reshape(_NW, nchunk, _CHUNK)
    out = _make_kernel(batch, nchunk)(emb_table, idx3)
    return out.reshape(*orig_shape, _HID)


# speedup vs baseline: 1.0842x; 1.0842x over previous
import functools

import jax
import jax.numpy as jnp
from jax import lax
from jax.experimental import pallas as pl
from jax.experimental.pallas import tpu as pltpu
from jax.experimental.pallas import tpu_sc as plsc

_HID = 64
_SCALE = 8.0

_NC = 2
_NS = 16
_NW = _NC * _NS
_LANES = 16

_CHUNK = 128
_VPC = _CHUNK * _HID // _LANES


_NBUF = 10
_LOOKAHEAD = 5


def _make_kernel(batch, nchunk):
    b_per_w = nchunk * _CHUNK
    nround = nchunk // _NBUF
    assert nround * _NBUF == nchunk
    mesh = plsc.VectorSubcoreMesh(
        core_axis_name="c", subcore_axis_name="s",
        num_cores=_NC, num_subcores=_NS,
    )

    @functools.partial(
        pl.kernel,
        out_type=jax.ShapeDtypeStruct((batch, _HID), jnp.float32),
        mesh=mesh,
        scratch_types=(
            [pltpu.VMEM((nchunk, _CHUNK), jnp.int32)]
            + [pltpu.VMEM((_CHUNK, _HID), jnp.float32) for _ in range(_NBUF)]
            + [pltpu.SemaphoreType.DMA for _ in range(2 * _NBUF)]
        ),
        compiler_params=pltpu.CompilerParams(use_tc_tiling_on_sc=False),
    )
    def emb_kernel(table_hbm, idx_hbm, out_hbm, idx_v, *scratch):
        rows = scratch[:_NBUF]
        gsem = scratch[_NBUF:2 * _NBUF]
        ssem = scratch[2 * _NBUF:]
        wid = lax.axis_index("s") * _NC + lax.axis_index("c")
        base = wid * b_per_w
        pltpu.sync_copy(idx_hbm.at[wid], idx_v)

        for b in range(_LOOKAHEAD):
            pltpu.async_copy(table_hbm.at[idx_v.at[b]], rows[b], gsem[b])

        def _scale(buf):
            @pl.loop(0, _CHUNK)
            def _rows(r):
                for c in range(_HID // _LANES):
                    sl = pl.ds(c * _LANES, _LANES)
                    buf[r, sl] = buf[r, sl] * _SCALE

        @pl.loop(0, nround)
        def _round(t):
            for b in range(_NBUF):
                g = t * _NBUF + b
                pb = (b + _LOOKAHEAD) % _NBUF
                pltpu.make_async_copy(
                    table_hbm.at[idx_v.at[g]], rows[b], gsem[b]).wait()

                def _prefetch(t=t, b=b, pb=pb):
                    gf = t * _NBUF + b + _LOOKAHEAD
                    gp = gf - _NBUF
                    pltpu.make_async_copy(
                        rows[pb],
                        out_hbm.at[pl.ds(base + gp * _CHUNK, _CHUNK)],
                        ssem[pb]).wait()
                    pltpu.async_copy(
                        table_hbm.at[idx_v.at[gf]], rows[pb], gsem[pb])

                if b < _LOOKAHEAD:
                    @pl.when(t >= 1)
                    def _(t=t, b=b, pb=pb):
                        _prefetch(t, b, pb)

                    @pl.when(t == 0)
                    def _(t=t, b=b, pb=pb):
                        gf = t * _NBUF + b + _LOOKAHEAD
                        pltpu.async_copy(
                            table_hbm.at[idx_v.at[gf]], rows[pb], gsem[pb])
                else:
                    @pl.when(t < nround - 1)
                    def _(t=t, b=b, pb=pb):
                        _prefetch(t, b, pb)

                _scale(rows[b])
                pltpu.async_copy(
                    rows[b],
                    out_hbm.at[pl.ds(base + g * _CHUNK, _CHUNK)],
                    ssem[b])

        for b in range(_NBUF):
            g = (nround - 1) * _NBUF + b
            pltpu.make_async_copy(
                rows[b],
                out_hbm.at[pl.ds(base + g * _CHUNK, _CHUNK)],
                ssem[b]).wait()

    return emb_kernel


def kernel(x, emb_table):
    orig_shape = x.shape
    idx = x.reshape(-1).astype(jnp.int32)
    batch = idx.shape[0]
    assert batch % (_NW * _CHUNK) == 0
    nchunk = batch // (_NW * _CHUNK)
    idx3 = idx.reshape(_NW, nchunk, _CHUNK)
    out = _make_kernel(batch, nchunk)(emb_table, idx3)
    return out.reshape(*orig_shape, _HID)
